# baseline (speedup 1.0000x reference)
import jax
import jax.numpy as jnp
from jax.experimental import pallas as pl

_K = 16
_G = 8


def _silu(t):
    return t / (1.0 + jnp.exp(-t))


def _rne(t):
    return t.astype(jnp.bfloat16).astype(jnp.float32)


def _body(cellrep_r, x_r, xT_r, z_r, emb_r, wm_r, bm_r, wu_r, bu_r, w1_r,
          b1_r, w2_r, b2_r, out_r):
    f32 = jnp.float32
    G = x_r.shape[0]
    n = x_r.shape[1]
    Gn = G * n
    L = wm_r.shape[0]
    F = emb_r.shape[1]

    xs = x_r[...].reshape(Gn, 3)
    frac = xs - jnp.floor(xs)
    xT = xT_r[...]
    fT = xT - jnp.floor(xT)
    cr = _rne(cellrep_r[...].reshape(Gn, 9))

    d = []
    for a in range(3):
        fTa = jnp.broadcast_to(fT[:, a:a + 1, :], (G, n, n)).reshape(Gn, n)
        t = frac[:, a:a + 1] - fTa
        t = t - jnp.round(t)
        d.append(_rne(t))
    cart = [d[0] * cr[:, 0 + c:1 + c] + d[1] * cr[:, 3 + c:4 + c]
            + d[2] * cr[:, 6 + c:7 + c] for c in range(3)]

    rloc = jax.lax.broadcasted_iota(jnp.int32, (G, n, n), 1).reshape(Gn, n)
    cI = jax.lax.broadcasted_iota(jnp.int32, (Gn, n), 1)
    colj = cI.astype(f32)
    dist2 = cart[0] * cart[0] + cart[1] * cart[1] + cart[2] * cart[2]
    D = dist2 + jnp.where(rloc == cI, 1e9, 0.0)

    P_list, d_list = [], []
    u_lists = [[], [], []]
    for _ in range(_K):
        m = jnp.min(D, axis=1, keepdims=True)
        am = jnp.min(jnp.where(D == m, colj, float(n)), axis=1,
                     keepdims=True)
        Pk = (colj == am).astype(f32)
        dk = jnp.sqrt(jnp.maximum(m, 1e-12))
        P_list.append(Pk)
        d_list.append(dk)
        inv = 1.0 / (dk + 1e-9)
        for c in range(3):
            u_lists[c].append(
                jnp.sum(Pk * cart[c], axis=1, keepdims=True) * inv)
        D = D + Pk * 1e9
    KH = _K // 2
    dlo_r = _rne(jnp.concatenate(d_list[:KH], axis=0))
    dhi_r = _rne(jnp.concatenate(d_list[KH:], axis=0))
    U = [jnp.concatenate(u_lists[c], axis=0) for c in range(3)]

    P_gs = [jnp.concatenate([P_list[k][g * n:(g + 1) * n, :]
                             for k in range(_K)], axis=0)
            for g in range(G)]

    zb = z_r[...].reshape(Gn, 1)
    lane = jax.lax.broadcasted_iota(jnp.int32, (Gn, emb_r.shape[0]),
                                    1).astype(f32)
    oh = (lane == zb).astype(f32)
    h = jnp.dot(oh, emb_r[...], preferred_element_type=f32,
                precision=jax.lax.Precision.HIGHEST)

    def edge_mlp(hcur, Wi, Wj, Wd, bv):
        hr = _rne(hcur)
        hwi = jnp.dot(hr, _rne(Wi), preferred_element_type=f32)
        hwj = jnp.dot(hr, _rne(Wj), preferred_element_type=f32)
        hj2_gs = []
        for g in range(G):
            hw_g = hwj[g * n:(g + 1) * n, :]
            lo = jnp.dot(P_gs[g][0:KH * n, :], hw_g,
                         preferred_element_type=f32,
                         precision=jax.lax.Precision.HIGHEST)
            hi = jnp.dot(P_gs[g][KH * n:, :], hw_g,
                         preferred_element_type=f32,
                         precision=jax.lax.Precision.HIGHEST)
            hj2_gs.append(jnp.concatenate([lo, hi], axis=1))
        hj2 = jnp.concatenate([hj2_gs[g][kp * n:(kp + 1) * n, :]
                               for kp in range(KH) for g in range(G)],
                              axis=0)
        hwi2 = jnp.concatenate([hwi, hwi], axis=1)
        hit2 = jnp.concatenate([hwi2] * KH, axis=0)
        wdr = _rne(Wd)
        dterm = jnp.concatenate([dlo_r * wdr, dhi_r * wdr], axis=1)
        bv2 = jnp.concatenate([bv, bv], axis=1)
        return _silu(hit2 + hj2 + dterm + bv2)

    for l in range(L):
        msg = edge_mlp(h, wm_r[l, 0:F, :], wm_r[l, F:2 * F, :],
                       wm_r[l, 2 * F:2 * F + 1, :], bm_r[l:l + 1, :])
        agg2 = msg[0:Gn, :]
        for kk in range(1, KH):
            agg2 = agg2 + msg[kk * Gn:(kk + 1) * Gn, :]
        agg = agg2[:, 0:F] + agg2[:, F:2 * F]
        upd = _silu(jnp.dot(_rne(h), _rne(wu_r[l, 0:F, :]),
                            preferred_element_type=f32)
                    + jnp.dot(_rne(agg), _rne(wu_r[l, F:2 * F, :]),
                              preferred_element_type=f32)
                    + bu_r[l:l + 1, :])
        h = h + upd

    u = edge_mlp(h, w1_r[0:F, :], w1_r[F:2 * F, :], w1_r[2 * F:2 * F + 1, :],
                 b1_r[...])
    ru = _rne(u)
    rw2 = _rne(w2_r[...])
    w_lo = jnp.sum(ru[:, 0:F] * rw2, axis=1, keepdims=True) + b2_r[0, 0]
    w_hi = jnp.sum(ru[:, F:2 * F] * rw2, axis=1, keepdims=True) + b2_r[0, 0]
    disp = []
    for c in range(3):
        t_lo = w_lo * U[c][0:KH * Gn, :]
        t_hi = w_hi * U[c][KH * Gn:, :]
        s = t_lo[0:Gn, :] + t_hi[0:Gn, :]
        for kk in range(1, KH):
            s = s + t_lo[kk * Gn:(kk + 1) * Gn, :]
            s = s + t_hi[kk * Gn:(kk + 1) * Gn, :]
        disp.append(s)
    out = frac + jnp.concatenate(disp, axis=1)
    out_r[...] = out.reshape(G, n, 3)


def kernel(cell, x, z, struct_size, emb, W_msg, b_msg, W_upd, b_upd,
           W1, b1, W2, b2):
    del struct_size
    B = cell.shape[0]
    N = x.shape[0]
    n = N // B
    F = emb.shape[1]
    HID = W1.shape[1]
    G = _G
    x3 = x.reshape(B, n, 3)
    xT3 = jnp.swapaxes(x3, 1, 2)
    zf = z.astype(jnp.float32).reshape(B, n, 1)
    cellrep = jnp.broadcast_to(cell.reshape(B, 1, 9), (B, n, 9))
    Vp = max(128, ((emb.shape[0] + 127) // 128) * 128)
    emb_p = jnp.zeros((Vp, F), jnp.float32).at[:emb.shape[0]].set(emb)
    b1r = b1.reshape(1, HID)
    w2r = W2.reshape(1, HID)
    b2r = b2.reshape(1, 1)

    out = pl.pallas_call(
        _body,
        grid=(B // G,),
        in_specs=[
            pl.BlockSpec((G, n, 9), lambda b: (b, 0, 0)),
            pl.BlockSpec((G, n, 3), lambda b: (b, 0, 0)),
            pl.BlockSpec((G, 3, n), lambda b: (b, 0, 0)),
            pl.BlockSpec((G, n, 1), lambda b: (b, 0, 0)),
            pl.BlockSpec((Vp, F), lambda b: (0, 0)),
            pl.BlockSpec(W_msg.shape, lambda b: (0, 0, 0)),
            pl.BlockSpec(b_msg.shape, lambda b: (0, 0)),
            pl.BlockSpec(W_upd.shape, lambda b: (0, 0, 0)),
            pl.BlockSpec(b_upd.shape, lambda b: (0, 0)),
            pl.BlockSpec(W1.shape, lambda b: (0, 0)),
            pl.BlockSpec((1, HID), lambda b: (0, 0)),
            pl.BlockSpec((1, HID), lambda b: (0, 0)),
            pl.BlockSpec((1, 1), lambda b: (0, 0)),
        ],
        out_specs=pl.BlockSpec((G, n, 3), lambda b: (b, 0, 0)),
        out_shape=jax.ShapeDtypeStruct((B, n, 3), jnp.float32),
    )(cellrep, x3, xT3, zf, emb_p, W_msg, b_msg, W_upd, b_upd, W1, b1r,
      w2r, b2r)
    return out.reshape(N, 3)


# speedup vs baseline: 1.2291x; 1.2291x over previous
import jax
import jax.numpy as jnp
from jax.experimental import pallas as pl

_K = 16
_G = 8


def _silu(t):
    return t / (1.0 + jnp.exp(-t))


def _rne(t):
    return t.astype(jnp.bfloat16).astype(jnp.float32)


def _body(cellrep_r, x_r, xT_r, z_r, emb_r, wm_r, bm_r, wu_r, bu_r, w1_r,
          b1_r, w2_r, b2_r, out_r):
    f32 = jnp.float32
    G = x_r.shape[0]
    n = x_r.shape[1]
    Gn = G * n
    L = wm_r.shape[0]
    F = emb_r.shape[1]

    xs = x_r[...].reshape(Gn, 3)
    frac = xs - jnp.floor(xs)
    xT = xT_r[...]
    fT = xT - jnp.floor(xT)
    cr = _rne(cellrep_r[...].reshape(Gn, 9))

    d = []
    for a in range(3):
        fTa = jnp.broadcast_to(fT[:, a:a + 1, :], (G, n, n)).reshape(Gn, n)
        t = frac[:, a:a + 1] - fTa
        t = t - jnp.round(t)
        d.append(_rne(t))
    cart = [d[0] * cr[:, 0 + c:1 + c] + d[1] * cr[:, 3 + c:4 + c]
            + d[2] * cr[:, 6 + c:7 + c] for c in range(3)]

    rloc = jax.lax.broadcasted_iota(jnp.int32, (G, n, n), 1).reshape(Gn, n)
    cI = jax.lax.broadcasted_iota(jnp.int32, (Gn, n), 1)
    colj = cI.astype(f32)
    dist2 = cart[0] * cart[0] + cart[1] * cart[1] + cart[2] * cart[2]
    D = dist2 + jnp.where(rloc == cI, 1e9, 0.0)

    P_list, d_list = [], []
    u_lists = [[], [], []]
    for _ in range(_K):
        m = jnp.min(D, axis=1, keepdims=True)
        am = jnp.min(jnp.where(D == m, colj, float(n)), axis=1,
                     keepdims=True)
        Pk = (colj == am).astype(f32)
        dk = jnp.sqrt(jnp.maximum(m, 1e-12))
        P_list.append(Pk)
        d_list.append(dk)
        inv = 1.0 / (dk + 1e-9)
        for c in range(3):
            u_lists[c].append(
                jnp.sum(Pk * cart[c], axis=1, keepdims=True) * inv)
        D = D + Pk * 1e9
    KH = _K // 2
    dlo_r = _rne(jnp.concatenate(d_list[:KH], axis=0))
    dhi_r = _rne(jnp.concatenate(d_list[KH:], axis=0))
    U = [jnp.concatenate(u_lists[c], axis=0) for c in range(3)]

    P_gs = [jnp.concatenate([P_list[k][g * n:(g + 1) * n, :]
                             for k in range(_K)], axis=0)
            for g in range(G)]

    zb = z_r[...].reshape(Gn, 1)
    lane = jax.lax.broadcasted_iota(jnp.int32, (Gn, emb_r.shape[0]),
                                    1).astype(f32)
    oh = (lane == zb).astype(f32)
    h = jnp.dot(oh, emb_r[...], preferred_element_type=f32,
                precision=jax.lax.Precision.HIGHEST)

    def edge_mlp(hcur, Wi, Wj, Wd, bv):
        hr = _rne(hcur)
        hwi = jnp.dot(hr, _rne(Wi), preferred_element_type=f32)
        hwj = jnp.dot(hr, _rne(Wj), preferred_element_type=f32)
        hj_gs = [jnp.dot(P_gs[g], hwj[g * n:(g + 1) * n, :],
                         preferred_element_type=f32,
                         precision=jax.lax.Precision.HIGHEST)
                 for g in range(G)]
        hj = jnp.concatenate([hj_gs[g][k * n:(k + 1) * n, :]
                              for k in range(_K) for g in range(G)], axis=0)
        hit = jnp.concatenate([hwi] * _K, axis=0)
        wdr = _rne(Wd)
        dterm = jnp.concatenate([dlo_r * wdr, dhi_r * wdr], axis=0)
        return _silu(hit + hj + dterm + bv)

    for l in range(L):
        msg = edge_mlp(h, wm_r[l, 0:F, :], wm_r[l, F:2 * F, :],
                       wm_r[l, 2 * F:2 * F + 1, :], bm_r[l:l + 1, :])
        agg = msg[0:Gn, :]
        for kk in range(1, _K):
            agg = agg + msg[kk * Gn:(kk + 1) * Gn, :]
        upd = _silu(jnp.dot(_rne(h), _rne(wu_r[l, 0:F, :]),
                            preferred_element_type=f32)
                    + jnp.dot(_rne(agg), _rne(wu_r[l, F:2 * F, :]),
                              preferred_element_type=f32)
                    + bu_r[l:l + 1, :])
        h = h + upd

    u = edge_mlp(h, w1_r[0:F, :], w1_r[F:2 * F, :], w1_r[2 * F:2 * F + 1, :],
                 b1_r[...])
    w = (jnp.sum(_rne(u) * _rne(w2_r[...]), axis=1, keepdims=True)
         + b2_r[0, 0])
    disp = []
    for c in range(3):
        t = w * U[c]
        s = t[0:Gn, :]
        for kk in range(1, _K):
            s = s + t[kk * Gn:(kk + 1) * Gn, :]
        disp.append(s)
    out = frac + jnp.concatenate(disp, axis=1)
    out_r[...] = out.reshape(G, n, 3)


def kernel(cell, x, z, struct_size, emb, W_msg, b_msg, W_upd, b_upd,
           W1, b1, W2, b2):
    del struct_size
    B = cell.shape[0]
    N = x.shape[0]
    n = N // B
    F = emb.shape[1]
    HID = W1.shape[1]
    G = _G
    x3 = x.reshape(B, n, 3)
    xT3 = jnp.swapaxes(x3, 1, 2)
    zf = z.astype(jnp.float32).reshape(B, n, 1)
    cellrep = jnp.broadcast_to(cell.reshape(B, 1, 9), (B, n, 9))
    Vp = max(128, ((emb.shape[0] + 127) // 128) * 128)
    emb_p = jnp.zeros((Vp, F), jnp.float32).at[:emb.shape[0]].set(emb)
    b1r = b1.reshape(1, HID)
    w2r = W2.reshape(1, HID)
    b2r = b2.reshape(1, 1)

    out = pl.pallas_call(
        _body,
        grid=(B // G,),
        in_specs=[
            pl.BlockSpec((G, n, 9), lambda b: (b, 0, 0)),
            pl.BlockSpec((G, n, 3), lambda b: (b, 0, 0)),
            pl.BlockSpec((G, 3, n), lambda b: (b, 0, 0)),
            pl.BlockSpec((G, n, 1), lambda b: (b, 0, 0)),
            pl.BlockSpec((Vp, F), lambda b: (0, 0)),
            pl.BlockSpec(W_msg.shape, lambda b: (0, 0, 0)),
            pl.BlockSpec(b_msg.shape, lambda b: (0, 0)),
            pl.BlockSpec(W_upd.shape, lambda b: (0, 0, 0)),
            pl.BlockSpec(b_upd.shape, lambda b: (0, 0)),
            pl.BlockSpec(W1.shape, lambda b: (0, 0)),
            pl.BlockSpec((1, HID), lambda b: (0, 0)),
            pl.BlockSpec((1, HID), lambda b: (0, 0)),
            pl.BlockSpec((1, 1), lambda b: (0, 0)),
        ],
        out_specs=pl.BlockSpec((G, n, 3), lambda b: (b, 0, 0)),
        out_shape=jax.ShapeDtypeStruct((B, n, 3), jnp.float32),
    )(cellrep, x3, xT3, zf, emb_p, W_msg, b_msg, W_upd, b_upd, W1, b1r,
      w2r, b2r)
    return out.reshape(N, 3)
